# baseline (speedup 1.0000x reference)
import functools

import jax
import jax.numpy as jnp
from jax import lax
from jax.experimental import pallas as pl
from jax.experimental.pallas import tpu as pltpu
from jax.experimental.pallas import tpu_sc as plsc

BATCH = 16384
FEAT_DIM = 64
LANES = 16
NUM_ROWS = 1000000
ROW_PAD = 128

_info = plsc.get_sparse_core_info()
NUM_CORES = _info.num_cores
NUM_SUBCORES = _info.num_subcores
NUM_WORKERS = NUM_CORES * NUM_SUBCORES
B_PER_W = BATCH // NUM_WORKERS
CHUNK = 64

RELAYOUT_BLOCK = 8000


def _relayout_body(in_ref, out_ref):
    x = in_ref[...]
    out_ref[:, 0:FEAT_DIM] = x
    out_ref[:, FEAT_DIM:ROW_PAD] = x


def _relayout_tc(table):
    return pl.pallas_call(
        _relayout_body,
        grid=(NUM_ROWS // RELAYOUT_BLOCK,),
        in_specs=[pl.BlockSpec((RELAYOUT_BLOCK, FEAT_DIM), lambda i: (i, 0))],
        out_specs=pl.BlockSpec((RELAYOUT_BLOCK, ROW_PAD), lambda i: (i, 0)),
        out_shape=jax.ShapeDtypeStruct((NUM_ROWS, ROW_PAD), jnp.float32),
    )(table)


def _sc_body(feat_hbm, y_hbm, table_hbm, out_hbm,
             y_v, rows_v, feat_v, out_v, sem):
    wid = lax.axis_index("s") * NUM_CORES + lax.axis_index("c")
    base = wid * B_PER_W

    pltpu.sync_copy(y_hbm.at[pl.ds(base, B_PER_W)], y_v)

    lane = lax.iota(jnp.int32, LANES)

    def chunk_body(c, carry):
        cbase = c * CHUNK
        gather = pltpu.async_copy(
            table_hbm.at[y_v.at[pl.ds(cbase, CHUNK)]], rows_v, sem)
        pltpu.sync_copy(feat_hbm.at[pl.ds(base + cbase, CHUNK)], feat_v)
        gather.wait()

        def group_body(g, carry2):
            outvec = jnp.zeros((LANES,), jnp.float32)
            for j in range(LANES):
                r = g * LANES + j
                acc = jnp.zeros((LANES,), jnp.float32)
                for q in range(FEAT_DIM // LANES):
                    f = feat_v[r, pl.ds(q * LANES, LANES)]
                    w = rows_v[r, pl.ds(q * LANES, LANES)]
                    acc = acc + f * w
                total = jnp.sum(acc)
                outvec = jnp.where(lane == j, total, outvec)
            out_v[pl.ds(cbase + g * LANES, LANES)] = outvec
            return carry2

        lax.fori_loop(0, CHUNK // LANES, group_body, 0)
        return carry

    lax.fori_loop(0, B_PER_W // CHUNK, chunk_body, 0)

    pltpu.sync_copy(out_v, out_hbm.at[pl.ds(base, B_PER_W)])


@jax.jit
def _projection_head(feat, y32, embed_weight):
    table_aligned = _relayout_tc(embed_weight)
    mesh = plsc.VectorSubcoreMesh(core_axis_name="c", subcore_axis_name="s")
    kern = functools.partial(
        pl.kernel,
        out_type=jax.ShapeDtypeStruct((BATCH,), jnp.float32),
        mesh=mesh,
        scratch_types=[
            pltpu.VMEM((B_PER_W,), jnp.int32),
            pltpu.VMEM((CHUNK, ROW_PAD), jnp.float32),
            pltpu.VMEM((CHUNK, FEAT_DIM), jnp.float32),
            pltpu.VMEM((B_PER_W,), jnp.float32),
            pltpu.SemaphoreType.DMA,
        ],
        compiler_params=pltpu.CompilerParams(needs_layout_passes=False),
    )(_sc_body)
    return kern(feat, y32, table_aligned)


def kernel(feat, y, embed_weight):
    return _projection_head(feat, y.astype(jnp.int32), embed_weight)
